# Initial kernel scaffold; baseline (speedup 1.0000x reference)
import jax
import jax.numpy as jnp
from jax.experimental import pallas as pl


def kernel(users, items, items_neg, edge_users, edge_items, user_embeds, item_embeds):
    raise NotImplementedError("write your pallas kernel here")


# speedup vs baseline: 5.8406x; 5.8406x over previous
import functools
import jax
import jax.numpy as jnp
from jax import lax
from jax.experimental import pallas as pl
from jax.experimental.pallas import tpu as pltpu
from jax.experimental.pallas import tpu_sc as plsc

NU = 50000
NI = 50000
E = 800000
D = 64
LAYERS = 3

HALF = 25000
ACC = 25088
RP = ACC // 16
K = 80
EPW = E // 16
NCHT = EPW // K
OUTER = 25
INNER = NCHT // OUTER
CW = 2 * K

_mesh = plsc.VectorSubcoreMesh(core_axis_name="c", subcore_axis_name="s")


@functools.partial(
    pl.kernel,
    out_type=jax.ShapeDtypeStruct((2 * ACC, D), jnp.float32),
    mesh=_mesh,
    compiler_params=pltpu.CompilerParams(use_tc_tiling_on_sc=False),
    scratch_types=[
        pltpu.VMEM((INNER * CW,), jnp.int32),
        pltpu.VMEM((K,), jnp.int32),
        pltpu.VMEM((K, D), jnp.float32),
        pltpu.VMEM_SHARED((ACC, D), jnp.float32),
        pltpu.SemaphoreType.DMA,
    ],
)
def _spmm(tab, comb, zeros, out, idx_v, dst_v, rows_v, acc, sem):
    c = lax.axis_index("c")
    s = lax.axis_index("s")
    base_row = s * RP

    pltpu.sync_copy(zeros, acc.at[pl.ds(base_row, RP)])
    plsc.subcore_barrier()

    def outer(o, carry):
        off = (c * (16 * NCHT) + s * NCHT + o * INNER) * CW
        pltpu.sync_copy(comb.at[pl.ds(off, INNER * CW)], idx_v)

        def inner(j, carry2):
            gidx = idx_v.at[pl.ds(j * CW, K)]
            pltpu.async_copy(tab.at[gidx], rows_v, sem).wait()
            for m in range(K // 16):
                dst_v[pl.ds(m * 16, 16)] = idx_v[pl.ds(j * CW + K + m * 16, 16)]
            pltpu.sync_copy(rows_v, acc.at[dst_v], add=True)
            return carry2

        lax.fori_loop(0, INNER, inner, 0)
        return carry

    lax.fori_loop(0, OUTER, outer, 0)
    plsc.subcore_barrier()
    pltpu.sync_copy(acc.at[pl.ds(base_row, RP)],
                    out.at[pl.ds(c * ACC + base_row, RP)])


def _build_comb(gather_idx, dst_idx):
    g = gather_idx.reshape(16 * NCHT, K)
    combs = []
    for c in range(2):
        local = dst_idx - c * HALF
        local = jnp.where((local >= 0) & (local < HALF), local, HALF)
        dl = local.reshape(16 * NCHT, K)
        combs.append(jnp.concatenate([g, dl], axis=1))
    return jnp.stack(combs).reshape(-1).astype(jnp.int32)


def _unpad(padded):
    return jnp.concatenate([padded[:HALF], padded[ACC:ACC + HALF]], axis=0)


def kernel(users, items, items_neg, edge_users, edge_items,
           user_embeds, item_embeds):
    eu = edge_users.astype(jnp.int32)
    ei = edge_items.astype(jnp.int32)

    u_deg = jnp.bincount(eu, length=NU)
    i_deg = jnp.bincount(ei, length=NI)
    u_norm = jnp.clip(u_deg, 1, None).astype(jnp.float32) ** -0.5
    i_norm = jnp.clip(i_deg, 1, None).astype(jnp.float32) ** -0.5

    comb_u = _build_comb(ei, eu)
    comb_i = _build_comb(eu, ei)
    zeros = jnp.zeros((RP, D), jnp.float32)

    ue = [user_embeds]
    ie = [item_embeds]
    for _ in range(LAYERS):
        nu = u_norm[:, None] * _unpad(_spmm(i_norm[:, None] * ie[-1],
                                            comb_u, zeros))
        ni = i_norm[:, None] * _unpad(_spmm(u_norm[:, None] * ue[-1],
                                            comb_i, zeros))
        ue.append(nu)
        ie.append(ni)

    final_u = sum(ue) / float(len(ue))
    final_i = sum(ie) / float(len(ie))

    u = final_u[users]
    it = final_i[items]
    it_neg = final_i[items_neg]
    pos = (u * it).sum(-1)
    neg = (u[:, None] * it_neg).sum(-1)
    return pos, neg


# speedup vs baseline: 7.5958x; 1.3005x over previous
import functools
import jax
import jax.numpy as jnp
from jax import lax
from jax.experimental import pallas as pl
from jax.experimental.pallas import tpu as pltpu
from jax.experimental.pallas import tpu_sc as plsc

NU = 50000
NI = 50000
E = 800000
D = 64
LAYERS = 3

HALF = 25000
ACC = 25088
RP = ACC // 16
K = 80
EPW = E // 16
NCHT = EPW // K
OUTER = 25
INNER = NCHT // OUTER
CW = 2 * K

_mesh = plsc.VectorSubcoreMesh(core_axis_name="c", subcore_axis_name="s")


@functools.partial(
    pl.kernel,
    out_type=jax.ShapeDtypeStruct((2 * ACC, D), jnp.float32),
    mesh=_mesh,
    compiler_params=pltpu.CompilerParams(use_tc_tiling_on_sc=False),
    scratch_types=[
        pltpu.VMEM((INNER * CW,), jnp.int32),
        pltpu.VMEM((K,), jnp.int32),
        pltpu.VMEM((K,), jnp.int32),
        pltpu.VMEM((K, D), jnp.float32),
        pltpu.VMEM((K, D), jnp.float32),
        pltpu.VMEM_SHARED((ACC, D), jnp.float32),
        pltpu.SemaphoreType.DMA,
        pltpu.SemaphoreType.DMA,
    ],
)
def _spmm(tab, comb, zeros, out, idx_v, dst0, dst1, rows0, rows1, acc,
          sem0, sem1):
    c = lax.axis_index("c")
    s = lax.axis_index("s")
    base_row = s * RP

    pltpu.sync_copy(zeros, acc.at[pl.ds(base_row, RP)])
    plsc.subcore_barrier()

    def gather_start(j, rows, sem):
        pltpu.async_copy(tab.at[idx_v.at[pl.ds(j * CW, K)]], rows, sem)

    def gather_wait(j, rows, sem):
        pltpu.make_async_copy(tab.at[idx_v.at[pl.ds(j * CW, K)]], rows,
                              sem).wait()

    def scatter(j, rows, dst):
        for m in range(K // 16):
            dst[pl.ds(m * 16, 16)] = idx_v[pl.ds(j * CW + K + m * 16, 16)]
        pltpu.sync_copy(rows, acc.at[dst], add=True)

    def outer(o, carry):
        off = (c * (16 * NCHT) + s * NCHT + o * INNER) * CW
        pltpu.sync_copy(comb.at[pl.ds(off, INNER * CW)], idx_v)
        gather_start(0, rows0, sem0)

        def pair(p, carry2):
            j0 = 2 * p
            gather_start(j0 + 1, rows1, sem1)
            gather_wait(j0, rows0, sem0)
            scatter(j0, rows0, dst0)
            gather_start(j0 + 2, rows0, sem0)
            gather_wait(j0 + 1, rows1, sem1)
            scatter(j0 + 1, rows1, dst1)
            return carry2

        lax.fori_loop(0, (INNER - 1) // 2, pair, 0)
        gather_wait(INNER - 1, rows0, sem0)
        scatter(INNER - 1, rows0, dst0)
        return carry

    lax.fori_loop(0, OUTER, outer, 0)
    plsc.subcore_barrier()
    pltpu.sync_copy(acc.at[pl.ds(base_row, RP)],
                    out.at[pl.ds(c * ACC + base_row, RP)])


def _build_comb(gather_idx, dst_idx):
    g = gather_idx.reshape(16 * NCHT, K)
    combs = []
    for c in range(2):
        local = dst_idx - c * HALF
        local = jnp.where((local >= 0) & (local < HALF), local, HALF)
        dl = local.reshape(16 * NCHT, K)
        combs.append(jnp.concatenate([g, dl], axis=1))
    return jnp.stack(combs).reshape(-1).astype(jnp.int32)


def _unpad(padded):
    return jnp.concatenate([padded[:HALF], padded[ACC:ACC + HALF]], axis=0)


def kernel(users, items, items_neg, edge_users, edge_items,
           user_embeds, item_embeds):
    eu = edge_users.astype(jnp.int32)
    ei = edge_items.astype(jnp.int32)

    u_deg = jnp.bincount(eu, length=NU)
    i_deg = jnp.bincount(ei, length=NI)
    u_norm = jnp.clip(u_deg, 1, None).astype(jnp.float32) ** -0.5
    i_norm = jnp.clip(i_deg, 1, None).astype(jnp.float32) ** -0.5

    comb_u = _build_comb(ei, eu)
    comb_i = _build_comb(eu, ei)
    zeros = jnp.zeros((RP, D), jnp.float32)

    ue = [user_embeds]
    ie = [item_embeds]
    for _ in range(LAYERS):
        nu = u_norm[:, None] * _unpad(_spmm(i_norm[:, None] * ie[-1],
                                            comb_u, zeros))
        ni = i_norm[:, None] * _unpad(_spmm(u_norm[:, None] * ue[-1],
                                            comb_i, zeros))
        ue.append(nu)
        ie.append(ni)

    final_u = sum(ue) / float(len(ue))
    final_i = sum(ie) / float(len(ie))

    u = final_u[users]
    it = final_i[items]
    it_neg = final_i[items_neg]
    pos = (u * it).sum(-1)
    neg = (u[:, None] * it_neg).sum(-1)
    return pos, neg
